# baseline (speedup 1.0000x reference)
import functools

import jax
import jax.numpy as jnp
from jax import lax
from jax.experimental import pallas as pl
from jax.experimental.pallas import tpu as pltpu
from jax.experimental.pallas import tpu_sc as plsc

N_NODES = 10000
N_EDGES = 320000
DIM = 128
N_GRAPHS = 64
N_LAYERS = 3

NC = 2
NS = 16
NW = NC * NS

CHUNK = 128
K = 2
N_CHUNKS = N_EDGES // CHUNK
N_GROUPS = N_CHUNKS // K
T_MAX = (N_GROUPS + NW - 1) // NW
N_PAD = 10240
ROWS_PER_TILE = N_PAD // NS


def _sc_aggregate(h, src2d, dst2d, zeros_tile):
    mesh = plsc.VectorSubcoreMesh(
        core_axis_name="c", subcore_axis_name="s", num_cores=NC, num_subcores=NS
    )

    @functools.partial(
        pl.kernel,
        out_type=jax.ShapeDtypeStruct((NC, N_PAD, DIM), jnp.float32),
        mesh=mesh,
        scratch_types=[
            pltpu.VMEM((K, CHUNK), jnp.int32),
            pltpu.VMEM((K, CHUNK), jnp.int32),
            pltpu.VMEM((K * CHUNK, DIM), jnp.float32),
            pltpu.VMEM_SHARED((N_PAD, DIM), jnp.float32),
            pltpu.SemaphoreType.DMA,
        ],
    )
    def body(h_hbm, src_hbm, dst_hbm, zero_hbm, out_hbm, src_v, dst_v, rows_v,
             acc_sh, sem):
        c = lax.axis_index("c")
        s = lax.axis_index("s")
        w = s * NC + c

        pltpu.sync_copy(zero_hbm, acc_sh.at[pl.ds(s * ROWS_PER_TILE, ROWS_PER_TILE)])
        plsc.subcore_barrier()

        def group_body(t, carry):
            g = w * T_MAX + t

            @pl.when(g < N_GROUPS)
            def _do():
                base = g * K
                pltpu.sync_copy(src_hbm.at[pl.ds(base, K)], src_v)
                pltpu.sync_copy(dst_hbm.at[pl.ds(base, K)], dst_v)
                descs = [
                    pltpu.async_copy(
                        h_hbm.at[src_v.at[k]],
                        rows_v.at[pl.ds(k * CHUNK, CHUNK)],
                        sem,
                    )
                    for k in range(K)
                ]
                for d in descs:
                    d.wait()
                for k in range(K):
                    pltpu.sync_copy(
                        rows_v.at[pl.ds(k * CHUNK, CHUNK)],
                        acc_sh.at[dst_v.at[k]],
                        add=True,
                    )

            return carry

        lax.fori_loop(0, T_MAX, group_body, 0)
        plsc.subcore_barrier()

        pltpu.sync_copy(
            acc_sh.at[pl.ds(s * ROWS_PER_TILE, ROWS_PER_TILE)],
            out_hbm.at[c, pl.ds(s * ROWS_PER_TILE, ROWS_PER_TILE)],
        )

    return body(h, src2d, dst2d, zeros_tile)


def _tc_layer(h, acc, epsp1, W1, b1, g1, be1, W2, b2, go, bo, batch2d):
    def body(eps_ref, h_ref, a_ref, w1_ref, b1_ref, g1_ref, be1_ref,
             w2_ref, b2_ref, go_ref, bo_ref, batch_ref, out_ref, pool_ref):
        ep = eps_ref[0]
        z = ep * h_ref[...] + a_ref[0, :N_NODES] + a_ref[1, :N_NODES]
        z1 = lax.dot_general(z, w1_ref[...], (((1,), (0,)), ((), ())),
                             preferred_element_type=jnp.float32) + b1_ref[...]
        mu1 = jnp.mean(z1, axis=0, keepdims=True)
        d1 = z1 - mu1
        var1 = jnp.mean(d1 * d1, axis=0, keepdims=True)
        z1n = jnp.maximum(
            g1_ref[...] * d1 * lax.rsqrt(var1 + 1e-5) + be1_ref[...], 0.0)
        z2 = lax.dot_general(z1n, w2_ref[...], (((1,), (0,)), ((), ())),
                             preferred_element_type=jnp.float32) + b2_ref[...]
        mu2 = jnp.mean(z2, axis=0, keepdims=True)
        d2 = z2 - mu2
        var2 = jnp.mean(d2 * d2, axis=0, keepdims=True)
        h_out = jnp.maximum(
            go_ref[...] * d2 * lax.rsqrt(var2 + 1e-5) + bo_ref[...], 0.0)
        out_ref[...] = h_out

        gids = lax.broadcasted_iota(jnp.int32, (N_GRAPHS, N_NODES), 0)
        onehot = (gids == batch_ref[...]).astype(jnp.float32)
        pool_ref[...] = lax.dot_general(
            onehot, h_out, (((1,), (0,)), ((), ())),
            preferred_element_type=jnp.float32)

    return pl.pallas_call(
        body,
        out_shape=(
            jax.ShapeDtypeStruct((N_NODES, DIM), jnp.float32),
            jax.ShapeDtypeStruct((N_GRAPHS, DIM), jnp.float32),
        ),
        in_specs=[pl.BlockSpec(memory_space=pltpu.SMEM)]
        + [pl.BlockSpec(memory_space=pltpu.VMEM)] * 11,
    )(epsp1, h, acc, W1, b1, g1, be1, W2, b2, go, bo, batch2d)


def kernel(x, edge_index, batch, eps, W1, b1, g1, be1, W2, b2, go, bo):
    src2d = edge_index[0].reshape(N_CHUNKS, CHUNK)
    dst2d = edge_index[1].reshape(N_CHUNKS, CHUNK)
    zeros_tile = jnp.zeros((ROWS_PER_TILE, DIM), jnp.float32)
    batch2d = batch.reshape(1, N_NODES)
    epsp1 = (1.0 + eps).astype(jnp.float32)

    h = x
    reps = []
    pooled = []
    for i in range(N_LAYERS):
        acc = _sc_aggregate(h, src2d, dst2d, zeros_tile)
        h, p = _tc_layer(
            h, acc, epsp1[i].reshape(1),
            W1[i], b1[i].reshape(1, DIM), g1[i].reshape(1, DIM),
            be1[i].reshape(1, DIM), W2[i], b2[i].reshape(1, DIM),
            go[i].reshape(1, DIM), bo[i].reshape(1, DIM), batch2d)
        reps.append(h)
        pooled.append(p)

    node_rep = jnp.concatenate(reps, axis=1)
    graph_rep = jnp.concatenate(pooled, axis=1)
    return (graph_rep, node_rep)


# speedup vs baseline: 1.0080x; 1.0080x over previous
import functools

import jax
import jax.numpy as jnp
from jax import lax
from jax.experimental import pallas as pl
from jax.experimental.pallas import tpu as pltpu
from jax.experimental.pallas import tpu_sc as plsc

N_NODES = 10000
N_EDGES = 320000
DIM = 128
N_GRAPHS = 64
N_LAYERS = 3

NC = 2
NS = 16
NW = NC * NS

CHUNK = 128
K = 2
E_PAD = 327680
N_CHUNKS = E_PAD // CHUNK
N_GROUPS = N_CHUNKS // K
T_MAX = N_GROUPS // NW
N_PAD = 10240
ROWS_PER_TILE = N_PAD // NS


def _sc_aggregate(h, src2d, dst2d, zeros_tile):
    mesh = plsc.VectorSubcoreMesh(
        core_axis_name="c", subcore_axis_name="s", num_cores=NC, num_subcores=NS
    )

    @functools.partial(
        pl.kernel,
        out_type=jax.ShapeDtypeStruct((NC, N_PAD, DIM), jnp.float32),
        mesh=mesh,
        scratch_types=[
            pltpu.VMEM((K, CHUNK), jnp.int32),
            pltpu.VMEM((K, CHUNK), jnp.int32),
            pltpu.VMEM((K * CHUNK, DIM), jnp.float32),
            pltpu.VMEM_SHARED((N_PAD, DIM), jnp.float32),
            pltpu.SemaphoreType.DMA,
        ],
    )
    def body(h_hbm, src_hbm, dst_hbm, zero_hbm, out_hbm, src_v, dst_v, rows_v,
             acc_sh, sem):
        c = lax.axis_index("c")
        s = lax.axis_index("s")
        w = s * NC + c

        pltpu.sync_copy(zero_hbm, acc_sh.at[pl.ds(s * ROWS_PER_TILE, ROWS_PER_TILE)])
        plsc.subcore_barrier()

        def group_body(t, carry):
            g = w * T_MAX + t

            @pl.when(g < N_GROUPS)
            def _do():
                base = g * K
                pltpu.sync_copy(src_hbm.at[pl.ds(base, K)], src_v)
                pltpu.sync_copy(dst_hbm.at[pl.ds(base, K)], dst_v)
                descs = [
                    pltpu.async_copy(
                        h_hbm.at[src_v.at[k]],
                        rows_v.at[pl.ds(k * CHUNK, CHUNK)],
                        sem,
                    )
                    for k in range(K)
                ]
                for d in descs:
                    d.wait()
                for k in range(K):
                    pltpu.sync_copy(
                        rows_v.at[pl.ds(k * CHUNK, CHUNK)],
                        acc_sh.at[dst_v.at[k]],
                        add=True,
                    )

            return carry

        lax.fori_loop(0, T_MAX, group_body, 0)
        plsc.subcore_barrier()

        pltpu.sync_copy(
            acc_sh.at[pl.ds(s * ROWS_PER_TILE, ROWS_PER_TILE)],
            out_hbm.at[c, pl.ds(s * ROWS_PER_TILE, ROWS_PER_TILE)],
        )

    return body(h, src2d, dst2d, zeros_tile)


def _tc_layer(h, acc, epsp1, W1, b1, g1, be1, W2, b2, go, bo, batch2d):
    def body(eps_ref, h_ref, a_ref, w1_ref, b1_ref, g1_ref, be1_ref,
             w2_ref, b2_ref, go_ref, bo_ref, batch_ref, out_ref, pool_ref):
        ep = eps_ref[0]
        z = ep * h_ref[...] + a_ref[0, :N_NODES] + a_ref[1, :N_NODES]
        z1 = lax.dot_general(z, w1_ref[...], (((1,), (0,)), ((), ())),
                             preferred_element_type=jnp.float32) + b1_ref[...]
        mu1 = jnp.mean(z1, axis=0, keepdims=True)
        d1 = z1 - mu1
        var1 = jnp.mean(d1 * d1, axis=0, keepdims=True)
        z1n = jnp.maximum(
            g1_ref[...] * d1 * lax.rsqrt(var1 + 1e-5) + be1_ref[...], 0.0)
        z2 = lax.dot_general(z1n, w2_ref[...], (((1,), (0,)), ((), ())),
                             preferred_element_type=jnp.float32) + b2_ref[...]
        mu2 = jnp.mean(z2, axis=0, keepdims=True)
        d2 = z2 - mu2
        var2 = jnp.mean(d2 * d2, axis=0, keepdims=True)
        h_out = jnp.maximum(
            go_ref[...] * d2 * lax.rsqrt(var2 + 1e-5) + bo_ref[...], 0.0)
        out_ref[...] = h_out

        gids = lax.broadcasted_iota(jnp.int32, (N_GRAPHS, N_NODES), 0)
        onehot = (gids == batch_ref[...]).astype(jnp.float32)
        pool_ref[...] = lax.dot_general(
            onehot, h_out, (((1,), (0,)), ((), ())),
            preferred_element_type=jnp.float32)

    return pl.pallas_call(
        body,
        out_shape=(
            jax.ShapeDtypeStruct((N_NODES, DIM), jnp.float32),
            jax.ShapeDtypeStruct((N_GRAPHS, DIM), jnp.float32),
        ),
        in_specs=[pl.BlockSpec(memory_space=pltpu.SMEM)]
        + [pl.BlockSpec(memory_space=pltpu.VMEM)] * 11,
    )(epsp1, h, acc, W1, b1, g1, be1, W2, b2, go, bo, batch2d)


def kernel(x, edge_index, batch, eps, W1, b1, g1, be1, W2, b2, go, bo):
    n_fill = E_PAD - N_EDGES
    fill = jnp.arange(n_fill, dtype=jnp.int32)
    src_pad = jnp.concatenate([edge_index[0], fill % N_NODES])
    dst_pad = jnp.concatenate(
        [edge_index[1], N_NODES + fill % (N_PAD - N_NODES)])
    src2d = src_pad.reshape(N_CHUNKS, CHUNK)
    dst2d = dst_pad.reshape(N_CHUNKS, CHUNK)
    zeros_tile = jnp.zeros((ROWS_PER_TILE, DIM), jnp.float32)
    batch2d = batch.reshape(1, N_NODES)
    epsp1 = (1.0 + eps).astype(jnp.float32)

    h = x
    reps = []
    pooled = []
    for i in range(N_LAYERS):
        acc = _sc_aggregate(h, src2d, dst2d, zeros_tile)
        h, p = _tc_layer(
            h, acc, epsp1[i].reshape(1),
            W1[i], b1[i].reshape(1, DIM), g1[i].reshape(1, DIM),
            be1[i].reshape(1, DIM), W2[i], b2[i].reshape(1, DIM),
            go[i].reshape(1, DIM), bo[i].reshape(1, DIM), batch2d)
        reps.append(h)
        pooled.append(p)

    node_rep = jnp.concatenate(reps, axis=1)
    graph_rep = jnp.concatenate(pooled, axis=1)
    return (graph_rep, node_rep)


# speedup vs baseline: 1.2004x; 1.1909x over previous
import functools

import jax
import jax.numpy as jnp
from jax import lax
from jax.experimental import pallas as pl
from jax.experimental.pallas import tpu as pltpu
from jax.experimental.pallas import tpu_sc as plsc

N_NODES = 10000
N_EDGES = 320000
DIM = 128
N_GRAPHS = 64
N_LAYERS = 3

NC = 2
NS = 16
NW = NC * NS

CHUNK = 128
E_PAD = 327680
N_CHUNKS = E_PAD // CHUNK
N_CHUNKS_ALLOC = N_CHUNKS + 8
CPW = N_CHUNKS // NW
UNROLL = 8
N_BODY = CPW // UNROLL
N_PAD = 10240
ROWS_PER_TILE = N_PAD // NS


def _sc_aggregate(h, src2d, dst2d, zeros_tile):
    mesh = plsc.VectorSubcoreMesh(
        core_axis_name="c", subcore_axis_name="s", num_cores=NC, num_subcores=NS
    )

    @functools.partial(
        pl.kernel,
        out_type=jax.ShapeDtypeStruct((NC, N_PAD, DIM), jnp.float32),
        mesh=mesh,
        scratch_types=[
            pltpu.VMEM((UNROLL, CHUNK), jnp.int32),
            pltpu.VMEM((UNROLL, CHUNK), jnp.int32),
            pltpu.VMEM((2 * CHUNK, DIM), jnp.float32),
            pltpu.VMEM_SHARED((N_PAD, DIM), jnp.float32),
            pltpu.SemaphoreType.DMA,
            pltpu.SemaphoreType.DMA,
        ],
    )
    def body(h_hbm, src_hbm, dst_hbm, zero_hbm, out_hbm, sidx, didx, rows,
             acc_sh, gsem, isem):
        c = lax.axis_index("c")
        s = lax.axis_index("s")
        w = s * NC + c
        base_chunk = w * CPW
        half_rows = UNROLL // 2

        def fire_gather(b, r):
            return pltpu.async_copy(h_hbm.at[sidx.at[r]],
                                    rows.at[pl.ds(b * CHUNK, CHUNK)], gsem)

        def scatter(b, r):
            pltpu.sync_copy(rows.at[pl.ds(b * CHUNK, CHUNK)],
                            acc_sh.at[didx.at[r]], add=True)

        def fire_idx(chunk0, r0):
            return (
                pltpu.async_copy(src_hbm.at[pl.ds(chunk0, half_rows)],
                                 sidx.at[pl.ds(r0, half_rows)], isem),
                pltpu.async_copy(dst_hbm.at[pl.ds(chunk0, half_rows)],
                                 didx.at[pl.ds(r0, half_rows)], isem),
            )

        def run4(r0):
            d0 = fire_gather(0, r0)
            d1 = fire_gather(1, r0 + 1)
            d0.wait()
            d1.wait()
            scatter(0, r0)
            scatter(1, r0 + 1)
            d2 = fire_gather(0, r0 + 2)
            d3 = fire_gather(1, r0 + 3)
            d2.wait()
            d3.wait()
            scatter(0, r0 + 2)
            scatter(1, r0 + 3)

        pltpu.sync_copy(zero_hbm,
                        acc_sh.at[pl.ds(s * ROWS_PER_TILE, ROWS_PER_TILE)])
        plsc.subcore_barrier()

        pltpu.sync_copy(src_hbm.at[pl.ds(base_chunk, half_rows)],
                        sidx.at[pl.ds(0, half_rows)])
        pltpu.sync_copy(dst_hbm.at[pl.ds(base_chunk, half_rows)],
                        didx.at[pl.ds(0, half_rows)])

        def body_t(t, carry):
            c0 = base_chunk + t * UNROLL
            ia, ib = fire_idx(c0 + 4, half_rows)
            run4(0)
            ia.wait()
            ib.wait()
            ic, id_ = fire_idx(c0 + UNROLL, 0)
            run4(half_rows)
            ic.wait()
            id_.wait()
            return carry

        lax.fori_loop(0, N_BODY, body_t, 0)
        plsc.subcore_barrier()

        pltpu.sync_copy(
            acc_sh.at[pl.ds(s * ROWS_PER_TILE, ROWS_PER_TILE)],
            out_hbm.at[c, pl.ds(s * ROWS_PER_TILE, ROWS_PER_TILE)],
        )

    return body(h, src2d, dst2d, zeros_tile)


def _tc_layer(h, acc, epsp1, W1, b1, g1, be1, W2, b2, go, bo, batch2d):
    def body(eps_ref, h_ref, a_ref, w1_ref, b1_ref, g1_ref, be1_ref,
             w2_ref, b2_ref, go_ref, bo_ref, batch_ref, out_ref, pool_ref):
        ep = eps_ref[0]
        z = ep * h_ref[...] + a_ref[0, :N_NODES] + a_ref[1, :N_NODES]
        z1 = lax.dot_general(z, w1_ref[...], (((1,), (0,)), ((), ())),
                             preferred_element_type=jnp.float32) + b1_ref[...]
        mu1 = jnp.mean(z1, axis=0, keepdims=True)
        d1 = z1 - mu1
        var1 = jnp.mean(d1 * d1, axis=0, keepdims=True)
        z1n = jnp.maximum(
            g1_ref[...] * d1 * lax.rsqrt(var1 + 1e-5) + be1_ref[...], 0.0)
        z2 = lax.dot_general(z1n, w2_ref[...], (((1,), (0,)), ((), ())),
                             preferred_element_type=jnp.float32) + b2_ref[...]
        mu2 = jnp.mean(z2, axis=0, keepdims=True)
        d2 = z2 - mu2
        var2 = jnp.mean(d2 * d2, axis=0, keepdims=True)
        h_out = jnp.maximum(
            go_ref[...] * d2 * lax.rsqrt(var2 + 1e-5) + bo_ref[...], 0.0)
        out_ref[...] = h_out

        gids = lax.broadcasted_iota(jnp.int32, (N_GRAPHS, N_NODES), 0)
        onehot = (gids == batch_ref[...]).astype(jnp.float32)
        pool_ref[...] = lax.dot_general(
            onehot, h_out, (((1,), (0,)), ((), ())),
            preferred_element_type=jnp.float32)

    return pl.pallas_call(
        body,
        out_shape=(
            jax.ShapeDtypeStruct((N_NODES, DIM), jnp.float32),
            jax.ShapeDtypeStruct((N_GRAPHS, DIM), jnp.float32),
        ),
        in_specs=[pl.BlockSpec(memory_space=pltpu.SMEM)]
        + [pl.BlockSpec(memory_space=pltpu.VMEM)] * 11,
    )(epsp1, h, acc, W1, b1, g1, be1, W2, b2, go, bo, batch2d)


def kernel(x, edge_index, batch, eps, W1, b1, g1, be1, W2, b2, go, bo):
    n_fill = N_CHUNKS_ALLOC * CHUNK - N_EDGES
    fill = jnp.arange(n_fill, dtype=jnp.int32)
    src_pad = jnp.concatenate([edge_index[0], fill % N_NODES])
    dst_pad = jnp.concatenate(
        [edge_index[1], N_NODES + fill % (N_PAD - N_NODES)])
    src2d = src_pad.reshape(N_CHUNKS_ALLOC, CHUNK)
    dst2d = dst_pad.reshape(N_CHUNKS_ALLOC, CHUNK)
    zeros_tile = jnp.zeros((ROWS_PER_TILE, DIM), jnp.float32)
    batch2d = batch.reshape(1, N_NODES)
    epsp1 = (1.0 + eps).astype(jnp.float32)

    h = x
    reps = []
    pooled = []
    for i in range(N_LAYERS):
        acc = _sc_aggregate(h, src2d, dst2d, zeros_tile)
        h, p = _tc_layer(
            h, acc, epsp1[i].reshape(1),
            W1[i], b1[i].reshape(1, DIM), g1[i].reshape(1, DIM),
            be1[i].reshape(1, DIM), W2[i], b2[i].reshape(1, DIM),
            go[i].reshape(1, DIM), bo[i].reshape(1, DIM), batch2d)
        reps.append(h)
        pooled.append(p)

    node_rep = jnp.concatenate(reps, axis=1)
    graph_rep = jnp.concatenate(pooled, axis=1)
    return (graph_rep, node_rep)


# speedup vs baseline: 1.4540x; 1.2112x over previous
import functools

import jax
import jax.numpy as jnp
from jax import lax
from jax.experimental import pallas as pl
from jax.experimental.pallas import tpu as pltpu
from jax.experimental.pallas import tpu_sc as plsc

N_NODES = 10000
N_EDGES = 320000
DIM = 128
N_GRAPHS = 64
N_LAYERS = 3

NC = 2
NS = 16
NW = NC * NS

CHUNK = 128
E_PAD = 327680
N_CHUNKS = E_PAD // CHUNK
N_CHUNKS_ALLOC = N_CHUNKS + 8
CPW = N_CHUNKS // NW
UNROLL = 8
N_BODY = CPW // UNROLL
N_PAD = 10240
ROWS_PER_TILE = N_PAD // NS


def _sc_aggregate(h, src2d, dst2d, zeros_tile):
    mesh = plsc.VectorSubcoreMesh(
        core_axis_name="c", subcore_axis_name="s", num_cores=NC, num_subcores=NS
    )

    @functools.partial(
        pl.kernel,
        out_type=jax.ShapeDtypeStruct((NC, N_PAD, DIM), jnp.float32),
        mesh=mesh,
        scratch_types=[
            pltpu.VMEM((UNROLL, CHUNK), jnp.int32),
            pltpu.VMEM((UNROLL, CHUNK), jnp.int32),
            pltpu.VMEM((2 * CHUNK, DIM), jnp.float32),
            pltpu.VMEM_SHARED((N_PAD, DIM), jnp.float32),
            pltpu.SemaphoreType.DMA,
            pltpu.SemaphoreType.DMA,
        ],
    )
    def body(h_hbm, src_hbm, dst_hbm, zero_hbm, out_hbm, sidx, didx, rows,
             acc_sh, gsem, isem):
        c = lax.axis_index("c")
        s = lax.axis_index("s")
        w = s * NC + c
        base_chunk = w * CPW
        half_rows = UNROLL // 2

        def fire_gather(b, r):
            return pltpu.async_copy(h_hbm.at[sidx.at[r]],
                                    rows.at[pl.ds(b * CHUNK, CHUNK)], gsem)

        def scatter(b, r):
            pltpu.sync_copy(rows.at[pl.ds(b * CHUNK, CHUNK)],
                            acc_sh.at[didx.at[r]], add=True)

        def fire_idx(chunk0, r0):
            return (
                pltpu.async_copy(src_hbm.at[pl.ds(chunk0, half_rows)],
                                 sidx.at[pl.ds(r0, half_rows)], isem),
                pltpu.async_copy(dst_hbm.at[pl.ds(chunk0, half_rows)],
                                 didx.at[pl.ds(r0, half_rows)], isem),
            )

        def run4(r0):
            d0 = fire_gather(0, r0)
            d1 = fire_gather(1, r0 + 1)
            d0.wait()
            scatter(0, r0)
            d2 = fire_gather(0, r0 + 2)
            d1.wait()
            scatter(1, r0 + 1)
            d3 = fire_gather(1, r0 + 3)
            d2.wait()
            scatter(0, r0 + 2)
            d3.wait()
            scatter(1, r0 + 3)

        pltpu.sync_copy(zero_hbm,
                        acc_sh.at[pl.ds(s * ROWS_PER_TILE, ROWS_PER_TILE)])
        plsc.subcore_barrier()

        pltpu.sync_copy(src_hbm.at[pl.ds(base_chunk, half_rows)],
                        sidx.at[pl.ds(0, half_rows)])
        pltpu.sync_copy(dst_hbm.at[pl.ds(base_chunk, half_rows)],
                        didx.at[pl.ds(0, half_rows)])

        def body_t(t, carry):
            c0 = base_chunk + t * UNROLL
            ia, ib = fire_idx(c0 + 4, half_rows)
            run4(0)
            ia.wait()
            ib.wait()
            ic, id_ = fire_idx(c0 + UNROLL, 0)
            run4(half_rows)
            ic.wait()
            id_.wait()
            return carry

        lax.fori_loop(0, N_BODY, body_t, 0)
        plsc.subcore_barrier()

        pltpu.sync_copy(
            acc_sh.at[pl.ds(s * ROWS_PER_TILE, ROWS_PER_TILE)],
            out_hbm.at[c, pl.ds(s * ROWS_PER_TILE, ROWS_PER_TILE)],
        )

    return body(h, src2d, dst2d, zeros_tile)


def _tc_layer(h, acc, epsp1, W1, b1, g1, be1, W2, b2, go, bo, batch2d):
    def body(eps_ref, h_ref, a_ref, w1_ref, b1_ref, g1_ref, be1_ref,
             w2_ref, b2_ref, go_ref, bo_ref, batch_ref, out_ref, pool_ref):
        ep = eps_ref[0]
        z = ep * h_ref[...] + a_ref[0, :N_NODES] + a_ref[1, :N_NODES]
        z1 = lax.dot_general(z, w1_ref[...], (((1,), (0,)), ((), ())),
                             preferred_element_type=jnp.float32) + b1_ref[...]
        mu1 = jnp.mean(z1, axis=0, keepdims=True)
        d1 = z1 - mu1
        var1 = jnp.mean(d1 * d1, axis=0, keepdims=True)
        z1n = jnp.maximum(
            g1_ref[...] * d1 * lax.rsqrt(var1 + 1e-5) + be1_ref[...], 0.0)
        z2 = lax.dot_general(z1n, w2_ref[...], (((1,), (0,)), ((), ())),
                             preferred_element_type=jnp.float32) + b2_ref[...]
        mu2 = jnp.mean(z2, axis=0, keepdims=True)
        d2 = z2 - mu2
        var2 = jnp.mean(d2 * d2, axis=0, keepdims=True)
        h_out = jnp.maximum(
            go_ref[...] * d2 * lax.rsqrt(var2 + 1e-5) + bo_ref[...], 0.0)
        out_ref[...] = h_out

        gids = lax.broadcasted_iota(jnp.int32, (N_GRAPHS, N_NODES), 0)
        onehot = (gids == batch_ref[...]).astype(jnp.float32)
        pool_ref[...] = lax.dot_general(
            onehot, h_out, (((1,), (0,)), ((), ())),
            preferred_element_type=jnp.float32)

    return pl.pallas_call(
        body,
        out_shape=(
            jax.ShapeDtypeStruct((N_NODES, DIM), jnp.float32),
            jax.ShapeDtypeStruct((N_GRAPHS, DIM), jnp.float32),
        ),
        in_specs=[pl.BlockSpec(memory_space=pltpu.SMEM)]
        + [pl.BlockSpec(memory_space=pltpu.VMEM)] * 11,
    )(epsp1, h, acc, W1, b1, g1, be1, W2, b2, go, bo, batch2d)


def kernel(x, edge_index, batch, eps, W1, b1, g1, be1, W2, b2, go, bo):
    n_fill = N_CHUNKS_ALLOC * CHUNK - N_EDGES
    fill = jnp.arange(n_fill, dtype=jnp.int32)
    src_pad = jnp.concatenate([edge_index[0], fill % N_NODES])
    dst_pad = jnp.concatenate(
        [edge_index[1], N_NODES + fill % (N_PAD - N_NODES)])
    src2d = src_pad.reshape(N_CHUNKS_ALLOC, CHUNK)
    dst2d = dst_pad.reshape(N_CHUNKS_ALLOC, CHUNK)
    zeros_tile = jnp.zeros((ROWS_PER_TILE, DIM), jnp.float32)
    batch2d = batch.reshape(1, N_NODES)
    epsp1 = (1.0 + eps).astype(jnp.float32)

    h = x
    reps = []
    pooled = []
    for i in range(N_LAYERS):
        acc = _sc_aggregate(h, src2d, dst2d, zeros_tile)
        h, p = _tc_layer(
            h, acc, epsp1[i].reshape(1),
            W1[i], b1[i].reshape(1, DIM), g1[i].reshape(1, DIM),
            be1[i].reshape(1, DIM), W2[i], b2[i].reshape(1, DIM),
            go[i].reshape(1, DIM), bo[i].reshape(1, DIM), batch2d)
        reps.append(h)
        pooled.append(p)

    node_rep = jnp.concatenate(reps, axis=1)
    graph_rep = jnp.concatenate(pooled, axis=1)
    return (graph_rep, node_rep)
